# baseline (speedup 1.0000x reference)
import functools

import jax
import jax.numpy as jnp
from jax import lax
from jax.experimental import pallas as pl
from jax.experimental.pallas import tpu as pltpu
from jax.experimental.pallas import tpu_sc as plsc

_LANES = 16


def _make_gather(B, V2, D2):
    D = D2 // 2
    info = plsc.get_sparse_core_info()
    NC, NS = info.num_cores, info.num_subcores
    NW = NC * NS
    assert B % (_LANES * NW) == 0, (B, NW)
    b_per_w = B // NW
    C = 256
    n_chunks = b_per_w // C
    assert b_per_w % C == 0
    mesh = plsc.VectorSubcoreMesh(core_axis_name="c", subcore_axis_name="s")

    @functools.partial(
        pl.kernel,
        mesh=mesh,
        out_type=jax.ShapeDtypeStruct((B, D), jnp.float32),
        scratch_types=[
            pltpu.VMEM((b_per_w + _LANES,), jnp.int32),
            pltpu.VMEM((C,), jnp.int32),
            pltpu.VMEM((C, D2), jnp.float32),
            pltpu.VMEM((C, D), jnp.float32),
            pltpu.SemaphoreType.DMA,
        ],
    )
    def gather_kernel(idx_hbm, table_hbm, out_hbm, idx_v, gid_v, pair_v,
                      row_v, sem):
        wid = lax.axis_index("s") * NC + lax.axis_index("c")
        base = wid * b_per_w
        pltpu.sync_copy(idx_hbm.at[pl.ds(base, b_per_w)],
                        idx_v.at[pl.ds(0, b_per_w)])
        for chunk in range(n_chunks):
            cbase = chunk * C
            for t in range(C // _LANES):
                v = idx_v[pl.ds(cbase + t * _LANES, _LANES)]
                gid_v[pl.ds(t * _LANES, _LANES)] = (
                    lax.shift_right_logical(v, 1))
            pltpu.async_copy(table_hbm.at[gid_v], pair_v, sem).wait()

            def body(j, _):
                v = idx_v[pl.ds(cbase + j, _LANES)]
                off = lax.bitwise_and(v[0], jnp.int32(1)) * jnp.int32(D)
                for k in range(D // _LANES):
                    row_v[j, pl.ds(k * _LANES, _LANES)] = (
                        pair_v[j, pl.ds(off + k * _LANES, _LANES)])
                return _

            lax.fori_loop(0, C, body, 0, unroll=False)
            pltpu.sync_copy(row_v, out_hbm.at[pl.ds(base + cbase, C)])

    return gather_kernel


def kernel(state_index, expansion_matrix):
    (B,) = state_index.shape
    V, D = expansion_matrix.shape
    table2 = jnp.concatenate(
        [expansion_matrix[0::2], expansion_matrix[1::2]], axis=1)
    gather = _make_gather(B, V // 2, 2 * D)
    return gather(state_index.astype(jnp.int32), table2)


# speedup vs baseline: 32.2155x; 32.2155x over previous
import functools

import jax
import jax.numpy as jnp
from jax import lax
from jax.experimental import pallas as pl
from jax.experimental.pallas import tpu as pltpu
from jax.experimental.pallas import tpu_sc as plsc

_LANES = 16
_GRP = 8


def _make_gather(B, V, D):
    info = plsc.get_sparse_core_info()
    NC, NS = info.num_cores, info.num_subcores
    NW = NC * NS
    assert B % (8 * NW) == 0, (B, NW)
    b_per_w = B // NW
    C = 32
    n_chunks = b_per_w // C
    assert b_per_w % C == 0
    mesh = plsc.VectorSubcoreMesh(core_axis_name="c", subcore_axis_name="s")

    @functools.partial(
        pl.kernel,
        mesh=mesh,
        compiler_params=pltpu.CompilerParams(needs_layout_passes=False),
        out_type=jax.ShapeDtypeStruct((B, D), jnp.float32),
        scratch_types=[
            pltpu.VMEM((b_per_w + _LANES,), jnp.int32),
            pltpu.VMEM((2, C, _GRP, D), jnp.float32),
            pltpu.VMEM((C, D), jnp.float32),
            pltpu.SemaphoreType.DMA,
            pltpu.SemaphoreType.DMA,
        ],
    )
    def gather_kernel(idx_hbm, table_hbm, out_hbm, idx_s, grp_v, row_v,
                      sem0, sem1):
        wid = lax.axis_index("s") * NC + lax.axis_index("c")
        base = wid * b_per_w
        pltpu.sync_copy(idx_hbm.at[pl.ds(base, b_per_w)],
                        idx_s.at[pl.ds(0, b_per_w)])
        sems = (sem0, sem1)

        def fire(chunk, buf, sem):
            cbase = chunk * C

            def issue(j, _):
                v = idx_s[pl.ds(cbase + j, _LANES)]
                gid = lax.shift_right_logical(v[0], 3)
                pltpu.async_copy(
                    table_hbm.at[pl.ds(gid, 1)],
                    grp_v.at[buf, pl.ds(j, 1)],
                    sem,
                )
                return _

            lax.fori_loop(0, C, issue, 0, unroll=4)

        def drain(buf, sem):
            pltpu.make_async_copy(
                table_hbm.at[pl.ds(0, C)], grp_v.at[buf], sem
            ).wait()

        def select_and_store(chunk, buf):
            cbase = chunk * C

            def body(j, _):
                v = idx_s[pl.ds(cbase + j, _LANES)]
                r = lax.bitwise_and(v[0], jnp.int32(_GRP - 1))
                for k in range(D // _LANES):
                    row_v[j, pl.ds(k * _LANES, _LANES)] = (
                        grp_v[buf, j, r, pl.ds(k * _LANES, _LANES)])
                return _

            lax.fori_loop(0, C, body, 0, unroll=4)
            pltpu.sync_copy(row_v, out_hbm.at[pl.ds(base + cbase, C)])

        fire(0, 0, sems[0])
        for chunk in range(n_chunks):
            buf = chunk % 2
            if chunk + 1 < n_chunks:
                fire(chunk + 1, 1 - buf, sems[1 - buf])
            drain(buf, sems[buf])
            select_and_store(chunk, buf)

    return gather_kernel


def kernel(state_index, expansion_matrix):
    (B,) = state_index.shape
    V, D = expansion_matrix.shape
    table3 = expansion_matrix.reshape(V // _GRP, _GRP, D)
    gather = _make_gather(B, V, D)
    return gather(state_index.astype(jnp.int32), table3)


# speedup vs baseline: 36.3210x; 1.1274x over previous
import functools

import jax
import jax.numpy as jnp
from jax import lax
from jax.experimental import pallas as pl
from jax.experimental.pallas import tpu as pltpu
from jax.experimental.pallas import tpu_sc as plsc


def _make_gather(B, V, D):
    info = plsc.get_sparse_core_info()
    NC, NS = info.num_cores, info.num_subcores
    NW = NC * NS
    assert B % (8 * NW) == 0, (B, NW)
    b_per_w = B // NW
    mesh = plsc.VectorSubcoreMesh(core_axis_name="c", subcore_axis_name="s")

    @functools.partial(
        pl.kernel,
        mesh=mesh,
        compiler_params=pltpu.CompilerParams(needs_layout_passes=False),
        out_type=jax.ShapeDtypeStruct((B, 1, D), jnp.float32),
        scratch_types=[
            pltpu.VMEM((b_per_w,), jnp.int32),
            pltpu.VMEM((b_per_w, 1, D), jnp.float32),
            pltpu.SemaphoreType.DMA,
        ],
    )
    def gather_kernel(idx_hbm, table_hbm, out_hbm, idx_v, rows_v, sem):
        wid = lax.axis_index("s") * NC + lax.axis_index("c")
        base = wid * b_per_w
        pltpu.sync_copy(idx_hbm.at[pl.ds(base, b_per_w)], idx_v)
        pltpu.async_copy(table_hbm.at[idx_v], rows_v, sem).wait()
        pltpu.sync_copy(rows_v, out_hbm.at[pl.ds(base, b_per_w)])

    return gather_kernel


def kernel(state_index, expansion_matrix):
    (B,) = state_index.shape
    V, D = expansion_matrix.shape
    table3 = expansion_matrix.reshape(V, 1, D)
    gather = _make_gather(B, V, D)
    out = gather(state_index.astype(jnp.int32), table3)
    return out.reshape(B, D)
